# baseline (speedup 1.0000x reference)
import functools

import numpy as np
import jax
import jax.numpy as jnp
from jax import lax
from jax.experimental import pallas as pl
from jax.experimental.pallas import tpu as pltpu
from jax.experimental.pallas import tpu_sc as plsc

_N_LEVELS = 16
_N_FEATS = 4
_LOG2_HASHMAP = 19
_BASE_RES = 16
_P1 = np.int32(np.uint32(2654435761).astype(np.int32))
_P2 = np.int32(805459861)

_NC, _NS = 2, 16
_NW = _NC * _NS
_NPTS = 65536
_PPW = _NPTS // _NW
_C = 256
_NCH = _PPW // _C
_NG = _C // 16
_CPL = 8 * _C
_NIDX = _CPL // 128


def _level_params():
    params = []
    for lvl in range(_N_LEVELS):
        scale = float(np.power(np.float32(2), np.float32(lvl)) * np.float32(_BASE_RES) - np.float32(1.0))
        res = int(np.int32(np.ceil(np.float32(scale))) + 1)
        size = min((res ** 3 + 7) // 8 * 8, 1 << _LOG2_HASHMAP)
        stride = 1
        strides = []
        for _ in range(3):
            strides.append(stride)
            stride = (stride * res) % (1 << 32)
            if stride > size:
                break
        use_hash = size < stride
        while len(strides) < 3:
            strides.append(0)
        if use_hash:
            y_live = z_live = True
        else:
            y_live = (strides[1] % size) != 0
            z_live = (strides[2] % size) != 0
        params.append((np.float32(scale), res, size, use_hash, tuple(strides),
                       y_live, z_live))
    return params

_LEVELS = _level_params()


@functools.lru_cache(maxsize=1)
def _make_sc_call():
    mesh = plsc.VectorSubcoreMesh(core_axis_name="c", subcore_axis_name="s",
                                  num_cores=_NC, num_subcores=_NS)

    @functools.partial(
        pl.kernel,
        out_type=jax.ShapeDtypeStruct((_NPTS, _N_LEVELS * _N_FEATS), jnp.float32),
        mesh=mesh,
        compiler_params=pltpu.CompilerParams(needs_layout_passes=False,
                                             use_tc_tiling_on_sc=False),
        scratch_types=[
            pltpu.VMEM((_C, 3), jnp.float32),
            pltpu.VMEM((3 * _CPL,), jnp.int32),
            pltpu.VMEM((3 * _CPL,), jnp.float32),
            pltpu.VMEM((3 * _CPL, 2 * _N_FEATS), jnp.float32),
            pltpu.VMEM((_C, _N_LEVELS * _N_FEATS), jnp.float32),
            pltpu.SemaphoreType.DMA,
            pltpu.SemaphoreType.DMA,
        ],
    )
    def sc_kernel(ct_hbm, table_hbm, out_hbm, crd_v, idx_v, w_v,
                  rows_v, out_v, sem0, sem1):
        wid = lax.axis_index("s") * _NC + lax.axis_index("c")
        iota = lax.iota(jnp.int32, 16)
        sems = (sem0, sem1)

        def emit_a(l, b, k):
            scale, res, size, use_hash, strides, y_live, z_live = _LEVELS[l]
            mask = jnp.int32(size - 1)
            corners = [(bx, by, bz)
                       for bx in range(2)
                       for by in (range(2) if y_live else (0,))
                       for bz in (range(2) if z_live else (0,))]
            if True:
                pvec = k * 16 + iota
                x = plsc.load_gather(crd_v, [pvec, jnp.full((16,), 0, jnp.int32)])
                px = x * scale + np.float32(0.5)
                ix = px.astype(jnp.int32)
                fx = px - ix.astype(jnp.float32)
                gx = np.float32(1.0) - fx
                wx = (gx, fx)
                if y_live:
                    y = plsc.load_gather(crd_v, [pvec, jnp.full((16,), 1, jnp.int32)])
                    py = y * scale + np.float32(0.5)
                    iy = py.astype(jnp.int32)
                    fy = py - iy.astype(jnp.float32)
                    gy = np.float32(1.0) - fy
                    wp = (gx * gy, gx * fy, fx * gy, fx * fy)
                if z_live:
                    z = plsc.load_gather(crd_v, [pvec, jnp.full((16,), 2, jnp.int32)])
                    pz = z * scale + np.float32(0.5)
                    iz = pz.astype(jnp.int32)
                    fz = pz - iz.astype(jnp.float32)
                    gz = np.float32(1.0) - fz
                    wz = (gz, fz)

                offs, ws = [], []
                if use_hash:
                    v0 = iy * _P1
                    v1 = v0 + _P1
                    t0 = iz * _P2
                    t1 = t0 + _P2
                    u = (ix, ix + 1)
                    v = (v0, v1)
                    t = (t0, t1)
                    exy = {}
                    for bx, by, bz in corners:
                        if (bx, by) not in exy:
                            exy[(bx, by)] = u[bx] ^ v[by]
                        offs.append((exy[(bx, by)] ^ t[bz]) & mask)
                        ws.append(wp[(bx << 1) | by] * wz[bz])
                else:
                    st1, st2 = strides[1], strides[2]
                    lin = ix
                    if y_live and st1:
                        lin = lin + (iy << int(np.log2(st1)))
                    if z_live and st2:
                        lin = lin + (iz << int(np.log2(st2)))
                    for bx, by, bz in corners:
                        kcv = (bx + by * st1 + bz * st2) % (1 << 32)
                        kc = jnp.int32(np.uint32(kcv).astype(np.int32))
                        offs.append((lin + kc) & mask)
                        if y_live:
                            wc = wp[(bx << 1) | by]
                            if z_live:
                                wc = wc * wz[bz]
                        else:
                            wc = wx[bx]
                        ws.append(wc)

                for c in range(len(corners)):
                    base = b * _CPL + c * _C + k * 16
                    off = offs[c]
                    idx_v[pl.ds(base, 16)] = lax.shift_right_logical(off, 1)
                    wc = ws[c]
                    w_v[pl.ds(base, 16)] = jnp.where((off & 1) != 0, -wc, wc)

        def ncorn(l):
            _, _, _, _, _, y_live, z_live = _LEVELS[l]
            return 2 * (2 if y_live else 1) * (2 if z_live else 1)

        def fire(l, b):
            n = ncorn(l) * _C
            return [
                pltpu.async_copy(
                    table_hbm.at[idx_v.at[pl.ds(b * _CPL, n)]],
                    rows_v.at[pl.ds(b * _CPL, n)],
                    sems[l & 1])
            ]

        def emit_b(l, b, k):
            nc = ncorn(l)
            pvec = k * 16 + iota
            four = jnp.full((16,), 4, jnp.int32)
            zero = jnp.zeros((16,), jnp.int32)
            wcs = []
            rvecs = []
            hvecs = []
            for c in range(nc):
                base = b * _CPL + c * _C + k * 16
                wsgn = w_v[pl.ds(base, 16)]
                wcs.append(jnp.abs(wsgn))
                hvecs.append(jnp.where(wsgn < np.float32(0.0), four, zero))
                rvecs.append(base + iota)
            for f in range(_N_FEATS):
                acc = None
                for c in range(nc):
                    cvec = hvecs[c] if f == 0 else hvecs[c] + f
                    g = plsc.load_gather(rows_v, [rvecs[c], cvec])
                    acc = wcs[c] * g if acc is None else acc + wcs[c] * g
                col = jnp.full((16,), l * _N_FEATS + f, jnp.int32)
                plsc.store_scatter(out_v, [pvec, col], acc)

        def phase_a(l, b):
            @plsc.parallel_loop(0, _NG, 1, unroll=1)
            def _(k):
                emit_a(l, b, k)

        def merged(l, bl, l2, b2):
            @plsc.parallel_loop(0, _NG, 1, unroll=1)
            def _(k):
                emit_b(l, bl, k)
                if l2 is not None:
                    emit_a(l2, b2, k)

        def chunk_body(ch, carry):
            gbase = wid * _PPW + ch * _C
            pltpu.sync_copy(ct_hbm.at[pl.ds(gbase, _C)], crd_v)
            dmas = {}
            phase_a(0, 0)
            dmas[0] = fire(0, 0)
            phase_a(1, 1)
            dmas[1] = fire(1, 1)
            for l in range(_N_LEVELS):
                for dma in dmas.pop(l):
                    dma.wait()
                l2 = l + 2 if l + 2 < _N_LEVELS else None
                merged(l, l % 3, l2, (l + 2) % 3)
                if l2 is not None:
                    dmas[l2] = fire(l2, l2 % 3)
            pltpu.sync_copy(out_v, out_hbm.at[pl.ds(gbase, _C)])
            return carry

        lax.fori_loop(0, _NCH, chunk_body, 0)

    return sc_kernel


def kernel(coords, params):
    table = params.reshape(-1, 2 * _N_FEATS)
    return _make_sc_call()(coords.astype(jnp.float32), table)


# speedup vs baseline: 1.0998x; 1.0998x over previous
import functools

import numpy as np
import jax
import jax.numpy as jnp
from jax import lax
from jax.experimental import pallas as pl
from jax.experimental.pallas import tpu as pltpu
from jax.experimental.pallas import tpu_sc as plsc

_N_LEVELS = 16
_N_FEATS = 4
_LOG2_HASHMAP = 19
_BASE_RES = 16
_P1 = np.int32(np.uint32(2654435761).astype(np.int32))
_P2 = np.int32(805459861)

_NC, _NS = 2, 16
_NW = _NC * _NS
_NPTS = 65536
_PPW = _NPTS // _NW
_C = 256
_NCH = _PPW // _C
_NG = _C // 16
_CPL = 8 * _C
_NIDX = _CPL // 128


def _level_params():
    params = []
    for lvl in range(_N_LEVELS):
        scale = float(np.power(np.float32(2), np.float32(lvl)) * np.float32(_BASE_RES) - np.float32(1.0))
        res = int(np.int32(np.ceil(np.float32(scale))) + 1)
        size = min((res ** 3 + 7) // 8 * 8, 1 << _LOG2_HASHMAP)
        stride = 1
        strides = []
        for _ in range(3):
            strides.append(stride)
            stride = (stride * res) % (1 << 32)
            if stride > size:
                break
        use_hash = size < stride
        while len(strides) < 3:
            strides.append(0)
        if use_hash:
            y_live = z_live = True
        else:
            y_live = (strides[1] % size) != 0
            z_live = (strides[2] % size) != 0
        params.append((np.float32(scale), res, size, use_hash, tuple(strides),
                       y_live, z_live))
    return params

_LEVELS = _level_params()


@functools.lru_cache(maxsize=1)
def _make_sc_call():
    mesh = plsc.VectorSubcoreMesh(core_axis_name="c", subcore_axis_name="s",
                                  num_cores=_NC, num_subcores=_NS)

    @functools.partial(
        pl.kernel,
        out_type=jax.ShapeDtypeStruct((_NPTS, _N_LEVELS * _N_FEATS), jnp.float32),
        mesh=mesh,
        compiler_params=pltpu.CompilerParams(needs_layout_passes=False,
                                             use_tc_tiling_on_sc=False),
        scratch_types=[
            pltpu.VMEM((_C,), jnp.float32),
            pltpu.VMEM((_C,), jnp.float32),
            pltpu.VMEM((_C,), jnp.float32),
            pltpu.VMEM((3 * _CPL,), jnp.int32),
            pltpu.VMEM((3 * _CPL,), jnp.int32),
            pltpu.VMEM((3 * _CPL,), jnp.float32),
            pltpu.VMEM((3 * _CPL, 2 * _N_FEATS), jnp.float32),
            pltpu.VMEM((_C, _N_LEVELS * _N_FEATS), jnp.float32),
            pltpu.SemaphoreType.DMA,
            pltpu.SemaphoreType.DMA,
        ],
    )
    def sc_kernel(ct_hbm, table_hbm, out_hbm, x_v, y_v, z_v, idx_v, h_v, w_v,
                  rows_v, out_v, sem0, sem1):
        wid = lax.axis_index("s") * _NC + lax.axis_index("c")
        iota = lax.iota(jnp.int32, 16)
        sems = (sem0, sem1)

        def emit_a(l, b, k):
            scale, res, size, use_hash, strides, y_live, z_live = _LEVELS[l]
            mask = jnp.int32(size - 1)
            corners = [(bx, by, bz)
                       for bx in range(2)
                       for by in (range(2) if y_live else (0,))
                       for bz in (range(2) if z_live else (0,))]
            if True:
                s16 = pl.ds(k * 16, 16)
                x = x_v[s16]
                px = x * scale + np.float32(0.5)
                ix = px.astype(jnp.int32)
                fx = px - ix.astype(jnp.float32)
                gx = np.float32(1.0) - fx
                wx = (gx, fx)
                if y_live:
                    y = y_v[s16]
                    py = y * scale + np.float32(0.5)
                    iy = py.astype(jnp.int32)
                    fy = py - iy.astype(jnp.float32)
                    gy = np.float32(1.0) - fy
                    wp = (gx * gy, gx * fy, fx * gy, fx * fy)
                if z_live:
                    z = z_v[s16]
                    pz = z * scale + np.float32(0.5)
                    iz = pz.astype(jnp.int32)
                    fz = pz - iz.astype(jnp.float32)
                    gz = np.float32(1.0) - fz
                    wz = (gz, fz)

                offs, ws = [], []
                if use_hash:
                    v0 = iy * _P1
                    v1 = v0 + _P1
                    t0 = iz * _P2
                    t1 = t0 + _P2
                    u = (ix, ix + 1)
                    v = (v0, v1)
                    t = (t0, t1)
                    exy = {}
                    for bx, by, bz in corners:
                        if (bx, by) not in exy:
                            exy[(bx, by)] = u[bx] ^ v[by]
                        offs.append((exy[(bx, by)] ^ t[bz]) & mask)
                        ws.append(wp[(bx << 1) | by] * wz[bz])
                else:
                    st1, st2 = strides[1], strides[2]
                    lin = ix
                    if y_live and st1:
                        lin = lin + (iy << int(np.log2(st1)))
                    if z_live and st2:
                        lin = lin + (iz << int(np.log2(st2)))
                    for bx, by, bz in corners:
                        kcv = (bx + by * st1 + bz * st2) % (1 << 32)
                        kc = jnp.int32(np.uint32(kcv).astype(np.int32))
                        offs.append((lin + kc) & mask)
                        if y_live:
                            wc = wp[(bx << 1) | by]
                            if z_live:
                                wc = wc * wz[bz]
                        else:
                            wc = wx[bx]
                        ws.append(wc)

                for c in range(len(corners)):
                    base = b * _CPL + c * _C + k * 16
                    off = offs[c]
                    idx_v[pl.ds(base, 16)] = lax.shift_right_logical(off, 1)
                    h_v[pl.ds(base, 16)] = (off & 1) << 2
                    w_v[pl.ds(base, 16)] = ws[c]

        def ncorn(l):
            _, _, _, _, _, y_live, z_live = _LEVELS[l]
            return 2 * (2 if y_live else 1) * (2 if z_live else 1)

        def fire(l, b):
            n = ncorn(l) * _C
            return [
                pltpu.async_copy(
                    table_hbm.at[idx_v.at[pl.ds(b * _CPL, n)]],
                    rows_v.at[pl.ds(b * _CPL, n)],
                    sems[l & 1])
            ]

        def emit_b(l, b, k):
            nc = ncorn(l)
            pvec = k * 16 + iota
            wcs = []
            rvecs = []
            hvecs = []
            for c in range(nc):
                base = b * _CPL + c * _C + k * 16
                wcs.append(w_v[pl.ds(base, 16)])
                hvecs.append(h_v[pl.ds(base, 16)])
                rvecs.append(base + iota)
            for f in range(_N_FEATS):
                acc = None
                for c in range(nc):
                    cvec = hvecs[c] if f == 0 else hvecs[c] + f
                    g = plsc.load_gather(rows_v, [rvecs[c], cvec])
                    acc = wcs[c] * g if acc is None else acc + wcs[c] * g
                col = jnp.full((16,), l * _N_FEATS + f, jnp.int32)
                plsc.store_scatter(out_v, [pvec, col], acc)

        def phase_a(l, b):
            @plsc.parallel_loop(0, _NG, 1, unroll=1)
            def _(k):
                emit_a(l, b, k)

        def merged(l, bl, l2, b2):
            @plsc.parallel_loop(0, _NG, 1, unroll=1)
            def _(k):
                emit_b(l, bl, k)
                if l2 is not None:
                    emit_a(l2, b2, k)

        def chunk_body(ch, carry):
            gbase = wid * _PPW + ch * _C
            pltpu.sync_copy(ct_hbm.at[pl.ds(gbase, _C)], x_v)
            pltpu.sync_copy(ct_hbm.at[pl.ds(_NPTS + gbase, _C)], y_v)
            pltpu.sync_copy(ct_hbm.at[pl.ds(2 * _NPTS + gbase, _C)], z_v)
            dmas = {}
            phase_a(0, 0)
            dmas[0] = fire(0, 0)
            phase_a(1, 1)
            dmas[1] = fire(1, 1)
            for l in range(_N_LEVELS):
                for dma in dmas.pop(l):
                    dma.wait()
                l2 = l + 2 if l + 2 < _N_LEVELS else None
                merged(l, l % 3, l2, (l + 2) % 3)
                if l2 is not None:
                    dmas[l2] = fire(l2, l2 % 3)
            pltpu.sync_copy(out_v, out_hbm.at[pl.ds(gbase, _C)])
            return carry

        lax.fori_loop(0, _NCH, chunk_body, 0)

    return sc_kernel


def kernel(coords, params):
    ct = coords.astype(jnp.float32).T.reshape(-1)
    table = params.reshape(-1, 2 * _N_FEATS)
    return _make_sc_call()(ct, table)
